# baseline (speedup 1.0000x reference)
import functools

import jax
import jax.numpy as jnp
from jax import lax
from jax.experimental import pallas as pl
from jax.experimental.pallas import tpu as pltpu
from jax.experimental.pallas import tpu_sc as plsc

_V = 49408
_D = 768
_T = 77
_B = 1024
_NW = 32
_ROWS = _B * _T
_RPW = _ROWS // _NW
_CHUNK = 32
_NCH = _RPW // _CHUNK
_NSLOT = 5
_NBLK = 75 // _NSLOT
_LANES = 16
_DV = _D // _LANES

_mesh = plsc.VectorSubcoreMesh(core_axis_name="c", subcore_axis_name="s")


@functools.partial(
    pl.kernel,
    out_type=jax.ShapeDtypeStruct((_ROWS, _D), jnp.float32),
    mesh=_mesh,
    scratch_types=(
        [pltpu.VMEM((_RPW,), jnp.int32),
         pltpu.VMEM((4 * _D,), jnp.float32)]
        + [pltpu.VMEM((_CHUNK, _D), jnp.float32) for _ in range(_NSLOT)]
        + [pltpu.VMEM((_CHUNK,), jnp.int32) for _ in range(_NSLOT)]
        + [pltpu.SemaphoreType.DMA for _ in range(2 * _NSLOT)]
    ),
)
def _embed_sc(tok_ref, pos_ref, tab_ref, out_ref, idx_v, pos_v, *rest):
    bufs = rest[:_NSLOT]
    oidx = rest[_NSLOT:2 * _NSLOT]
    gsems = rest[2 * _NSLOT:3 * _NSLOT]
    ssems = rest[3 * _NSLOT:4 * _NSLOT]

    wid = lax.axis_index("s") * 2 + lax.axis_index("c")
    base = wid * _RPW
    t0 = base // _B

    pltpu.sync_copy(tok_ref.at[pl.ds(base, _RPW)], idx_v)
    pltpu.sync_copy(pos_ref.at[pl.ds(t0 * _D, 4 * _D)], pos_v)

    def fire_gather(k, sl):
        pltpu.async_copy(
            tab_ref.at[idx_v.at[pl.ds(k * _CHUNK, _CHUNK)]], bufs[sl],
            gsems[sl])

    def wait_gather(k, sl):
        pltpu.make_async_copy(
            tab_ref.at[idx_v.at[pl.ds(k * _CHUNK, _CHUNK)]], bufs[sl],
            gsems[sl]).wait()

    def wait_store(sl):
        pltpu.make_async_copy(bufs[sl], out_ref.at[oidx[sl]],
                              ssems[sl]).wait()

    def process(k, sl):
        g = base + k * _CHUNK
        t = g // _B
        b0 = g % _B
        ti = t - t0
        row0 = b0 * _T + t
        i16 = lax.iota(jnp.int32, _LANES) * _T
        oidx[sl][pl.ds(0, _LANES)] = i16 + row0
        oidx[sl][pl.ds(_LANES, _LANES)] = i16 + (row0 + _LANES * _T)

        wait_gather(k, sl)

        def dv_body(dv, carry):
            off = dv * _LANES
            pv = pos_v[pl.ds(ti * _D + off, _LANES)]
            for i in range(_CHUNK):
                plsc.addupdate(bufs[sl].at[i, pl.ds(off, _LANES)], pv)
            return carry
        lax.fori_loop(0, _DV, dv_body, 0)

        pltpu.async_copy(bufs[sl], out_ref.at[oidx[sl]], ssems[sl])

    for sl in range(_NSLOT):
        fire_gather(sl, sl)

    def block(o, carry):
        for s in range(_NSLOT):
            k = o * _NSLOT + s
            process(k, s)
            s3 = (s - 2) % _NSLOT

            @pl.when(jnp.logical_and(k >= 2, k + 3 <= _NCH - 1))
            def _retire_refill():
                wait_store(s3)
                fire_gather(k + 3, s3)

        return carry

    lax.fori_loop(0, _NBLK, block, 0)

    process(_NCH - 2, 0)
    process(_NCH - 1, 1)

    for sl in (2, 3, 4, 0, 1):
        wait_store(sl)


def kernel(tokens, token_embd, position_embd):
    tokens_t = tokens.astype(jnp.int32).T.reshape(-1)
    pos_flat = jnp.pad(position_embd, ((0, 3), (0, 0))).reshape(-1)
    out = _embed_sc(tokens_t, pos_flat, token_embd)
    return out.reshape(_B, _T, _D)
